# baseline (speedup 1.0000x reference)
import functools

import jax
import jax.numpy as jnp
from jax import lax
from jax.experimental import pallas as pl
from jax.experimental.pallas import tpu as pltpu
from jax.experimental.pallas import tpu_sc as plsc

N = 10000
E = 320000
D = 128
H = 2
C = 64
HC = H * C
EPS = 0.01
NEG_SLOPE = 0.2

NS = 16
EPT = E // NS
K = 80
NCHUNK = EPT // K
NP = 10240
ROWS_PT = NP // NS
AW = 80


def _sc_gat_edges(table_hbm, src_hbm, dst_hbm, attr_hbm, zeros_hbm, out_hbm,
                  idxs_v, idxd_v,
                  srca0, srca1, dsta0, dsta1, dstr0, dstr1,
                  xj0, xj1, xi0, xi1, msg0, msg1, attr_v, s16_v, accum,
                  sgj0, sgj1, sgi0, sgi1, ssc0, ssc1):
    cid = lax.axis_index("c")
    sid = lax.axis_index("s")
    srca = (srca0, srca1)
    dsta = (dsta0, dsta1)
    dstr = (dstr0, dstr1)
    xj = (xj0, xj1)
    xi = (xi0, xi1)
    msg = (msg0, msg1)
    sgj = (sgj0, sgj1)
    sgi = (sgi0, sgi1)
    ssc = (ssc0, ssc1)

    pltpu.sync_copy(zeros_hbm.at[pl.ds(sid * ROWS_PT, ROWS_PT)],
                    accum.at[pl.ds(sid * ROWS_PT, ROWS_PT)])
    pltpu.sync_copy(attr_hbm.at[cid], attr_v)
    tile_base = sid * EPT
    pltpu.sync_copy(src_hbm.at[pl.ds(tile_base, EPT)], idxs_v)
    pltpu.sync_copy(dst_hbm.at[pl.ds(tile_base, EPT)], idxd_v)
    plsc.subcore_barrier()

    row_off = cid * N
    attr_q = [attr_v[pl.ds(q * 16, 16)] for q in range(C // 16)]

    def adjust_and_gather(c, b):
        off = c * K
        for i in range(K // 16):
            dsl = pl.ds(off + i * 16, 16)
            sl = pl.ds(i * 16, 16)
            srca[b][sl] = idxs_v[dsl] + row_off
            d = idxd_v[dsl]
            dsta[b][sl] = d + row_off
            dstr[b][sl] = d
        pltpu.async_copy(table_hbm.at[srca[b]], xj[b], sgj[b])
        pltpu.async_copy(table_hbm.at[dsta[b]], xi[b], sgi[b])

    def wait_gather(b):
        pltpu.make_async_copy(table_hbm.at[srca[b]], xj[b], sgj[b]).wait()
        pltpu.make_async_copy(table_hbm.at[dsta[b]], xi[b], sgi[b]).wait()

    iota16 = lax.iota(jnp.int32, 16)
    ii16 = iota16 * 16

    def compute(b):
        xj_r, xi_r, msg_r = xj[b], xi[b], msg[b]
        for g in range(K // 16):
            for k in range(16):
                e = g * 16 + k
                s = None
                for q in range(C // 16):
                    sl = pl.ds(q * 16, 16)
                    m = xj_r[e, sl] + (1.0 + EPS) * xi_r[e, sl]
                    lr = jnp.maximum(m, NEG_SLOPE * m)
                    t = lr * attr_q[q]
                    s = t if s is None else s + t
                s16_v[pl.ds(k * 16, 16)] = s
            av = None
            for l in range(16):
                v = plsc.load_gather(s16_v, [ii16 + l])
                av = v if av is None else av + v
            p_vec = jnp.exp(av)
            for k in range(16):
                e = g * 16 + k
                pb = jnp.full((16,), p_vec[k], dtype=jnp.float32)
                for q in range(C // 16):
                    sl = pl.ds(q * 16, 16)
                    msg_r[e, sl] = xj_r[e, sl] * pb
                msg_r[e, pl.ds(C, 16)] = pb

    def start_scatter(b):
        pltpu.async_copy(msg[b], accum.at[dstr[b]], ssc[b], add=True)

    def wait_scatter(b):
        pltpu.make_async_copy(msg[b], accum.at[dstr[b]], ssc[b]).wait()

    adjust_and_gather(0, 0)

    def super_body(s_, carry):
        for b in (0, 1):
            c = 2 * s_ + b
            nb = 1 - b
            wait_gather(b)

            @pl.when(c >= 1)
            def _wait_prev():
                wait_scatter(nb)

            @pl.when(c + 1 < NCHUNK)
            def _prefetch_next():
                adjust_and_gather(c + 1, nb)

            compute(b)
            start_scatter(b)
        return carry

    lax.fori_loop(0, NCHUNK // 2, super_body, 0)
    wait_scatter(1)

    plsc.subcore_barrier()
    pltpu.sync_copy(accum.at[pl.ds(sid * ROWS_PT, ROWS_PT)],
                    out_hbm.at[cid, pl.ds(sid * ROWS_PT, ROWS_PT)])


@functools.cache
def _sc_gat_kernel():
    return pl.kernel(
        _sc_gat_edges,
        out_type=jax.ShapeDtypeStruct((H, NP, AW), jnp.float32),
        mesh=plsc.VectorSubcoreMesh(core_axis_name="c", subcore_axis_name="s",
                                    num_cores=H, num_subcores=NS),
        compiler_params=pltpu.CompilerParams(use_tc_tiling_on_sc=False,
                                             needs_layout_passes=False),
        scratch_types=[
            pltpu.VMEM((EPT,), jnp.int32),
            pltpu.VMEM((EPT,), jnp.int32),
            pltpu.VMEM((K,), jnp.int32),
            pltpu.VMEM((K,), jnp.int32),
            pltpu.VMEM((K,), jnp.int32),
            pltpu.VMEM((K,), jnp.int32),
            pltpu.VMEM((K,), jnp.int32),
            pltpu.VMEM((K,), jnp.int32),
            pltpu.VMEM((K, C), jnp.float32),
            pltpu.VMEM((K, C), jnp.float32),
            pltpu.VMEM((K, C), jnp.float32),
            pltpu.VMEM((K, C), jnp.float32),
            pltpu.VMEM((K, AW), jnp.float32),
            pltpu.VMEM((K, AW), jnp.float32),
            pltpu.VMEM((C,), jnp.float32),
            pltpu.VMEM((256,), jnp.float32),
            pltpu.VMEM_SHARED((NP, AW), jnp.float32),
            pltpu.SemaphoreType.DMA,
            pltpu.SemaphoreType.DMA,
            pltpu.SemaphoreType.DMA,
            pltpu.SemaphoreType.DMA,
            pltpu.SemaphoreType.DMA,
            pltpu.SemaphoreType.DMA,
        ],
    )


def _tc_in_body(x_ref, w_ref, b_ref, out_ref):
    y = jnp.dot(x_ref[...], w_ref[...].T, preferred_element_type=jnp.float32)
    y = y + b_ref[...]
    out_ref[pl.ds(0, N), :] = y[:, 0:C]
    out_ref[pl.ds(N, N), :] = y[:, C:HC]


def _tc_mid_body(acc_ref, w_ref, b_ref, out_ref):
    num0 = acc_ref[0, 0:N, 0:C]
    den0 = acc_ref[0, 0:N, C:C + 1]
    num1 = acc_ref[1, 0:N, 0:C]
    den1 = acc_ref[1, 0:N, C:C + 1]
    h0 = jnp.maximum(num0 / (den0 + 1e-16), 0.0)
    h1 = jnp.maximum(num1 / (den1 + 1e-16), 0.0)
    h = jnp.concatenate([h0, h1], axis=1)
    y = jnp.dot(h, w_ref[...].T, preferred_element_type=jnp.float32)
    y = y + b_ref[...]
    out_ref[pl.ds(0, N), :] = y[:, 0:C]
    out_ref[pl.ds(N, N), :] = y[:, C:HC]


def _tc_head_body(acc_ref, w1_ref, b1_ref, w2_ref, b2_ref, out_ref):
    num0 = acc_ref[0, 0:N, 0:C]
    den0 = acc_ref[0, 0:N, C:C + 1]
    num1 = acc_ref[1, 0:N, 0:C]
    den1 = acc_ref[1, 0:N, C:C + 1]
    h0 = jnp.maximum(num0 / (den0 + 1e-16), 0.0)
    h1 = jnp.maximum(num1 / (den1 + 1e-16), 0.0)
    h = jnp.concatenate([h0, h1], axis=1)
    y = jnp.dot(h, w1_ref[...].T, preferred_element_type=jnp.float32)
    y = y + b1_ref[...]
    y = jnp.dot(y, w2_ref[...].T, preferred_element_type=jnp.float32)
    y = y + b2_ref[...]
    m = jnp.max(y, axis=1, keepdims=True)
    z = y - m
    out_ref[...] = z - jnp.log(jnp.sum(jnp.exp(z), axis=1, keepdims=True))


_tc_in = pl.pallas_call(
    _tc_in_body,
    out_shape=jax.ShapeDtypeStruct((H * N, C), jnp.float32),
)

_tc_mid = pl.pallas_call(
    _tc_mid_body,
    out_shape=jax.ShapeDtypeStruct((H * N, C), jnp.float32),
)

_tc_head = pl.pallas_call(
    _tc_head_body,
    out_shape=jax.ShapeDtypeStruct((N, C), jnp.float32),
)


def kernel(x, edge_index, batch, W0, b0, attr0, W1, b1, attr1, Wp1, bp1, Wp2, bp2):
    src = edge_index[0]
    dst = edge_index[1]
    zeros = jnp.zeros((NP, AW), dtype=jnp.float32)
    attr0f = attr0.reshape(H, C)
    attr1f = attr1.reshape(H, C)

    sc_gat = _sc_gat_kernel()
    table0 = _tc_in(x, W0, b0.reshape(1, HC))
    acc0 = sc_gat(table0, src, dst, attr0f, zeros)
    table1 = _tc_mid(acc0, W1, b1.reshape(1, HC))
    acc1 = sc_gat(table1, src, dst, attr1f, zeros)
    return _tc_head(acc1, Wp1, bp1.reshape(1, C), Wp2, bp2.reshape(1, C))


# speedup vs baseline: 1.3334x; 1.3334x over previous
import functools

import jax
import jax.numpy as jnp
from jax import lax
from jax.experimental import pallas as pl
from jax.experimental.pallas import tpu as pltpu
from jax.experimental.pallas import tpu_sc as plsc

N = 10000
E = 320000
D = 128
H = 2
C = 64
HC = H * C
EPS = 0.01
NEG_SLOPE = 0.2

NS = 16
EPT = E // NS
K = 80
NCHUNK = EPT // K
NP = 10240
ROWS_PT = NP // NS
AW = 80


def _sc_gat_edges(table_hbm, src_hbm, dst_hbm, attr_hbm, zeros_hbm, out_hbm,
                  idxs_v, idxd_v,
                  srca0, srca1, dsta0, dsta1, dstr0, dstr1,
                  xj0, xj1, xi0, xi1, msg0, msg1, attr_v, accum,
                  sgj0, sgj1, sgi0, sgi1, ssc0, ssc1):
    cid = lax.axis_index("c")
    sid = lax.axis_index("s")
    srca = (srca0, srca1)
    dsta = (dsta0, dsta1)
    dstr = (dstr0, dstr1)
    xj = (xj0, xj1)
    xi = (xi0, xi1)
    msg = (msg0, msg1)
    sgj = (sgj0, sgj1)
    sgi = (sgi0, sgi1)
    ssc = (ssc0, ssc1)

    pltpu.sync_copy(zeros_hbm.at[pl.ds(sid * ROWS_PT, ROWS_PT)],
                    accum.at[pl.ds(sid * ROWS_PT, ROWS_PT)])
    pltpu.sync_copy(attr_hbm.at[cid], attr_v)
    tile_base = sid * EPT
    pltpu.sync_copy(src_hbm.at[pl.ds(tile_base, EPT)], idxs_v)
    pltpu.sync_copy(dst_hbm.at[pl.ds(tile_base, EPT)], idxd_v)
    plsc.subcore_barrier()

    row_off = cid * N
    attr_q = [attr_v[pl.ds(q * 16, 16)] for q in range(C // 16)]

    def adjust_and_gather(c, b):
        off = c * K
        for i in range(K // 16):
            dsl = pl.ds(off + i * 16, 16)
            sl = pl.ds(i * 16, 16)
            srca[b][sl] = idxs_v[dsl] + row_off
            d = idxd_v[dsl]
            dsta[b][sl] = d + row_off
            dstr[b][sl] = d
        pltpu.async_copy(table_hbm.at[srca[b]], xj[b], sgj[b])
        pltpu.async_copy(table_hbm.at[dsta[b]], xi[b], sgi[b])

    def wait_gather(b):
        pltpu.make_async_copy(table_hbm.at[srca[b]], xj[b], sgj[b]).wait()
        pltpu.make_async_copy(table_hbm.at[dsta[b]], xi[b], sgi[b]).wait()

    def compute(b):
        xj_r, xi_r, msg_r = xj[b], xi[b], msg[b]
        for e in range(K):
            s = None
            for q in range(C // 16):
                sl = pl.ds(q * 16, 16)
                m = xj_r[e, sl] + (1.0 + EPS) * xi_r[e, sl]
                lr = jnp.maximum(m, NEG_SLOPE * m)
                t = lr * attr_q[q]
                s = t if s is None else s + t
            pb = jnp.exp(jnp.full((16,), jnp.sum(s), dtype=jnp.float32))
            for q in range(C // 16):
                sl = pl.ds(q * 16, 16)
                msg_r[e, sl] = xj_r[e, sl] * pb
            msg_r[e, pl.ds(C, 16)] = pb

    def start_scatter(b):
        pltpu.async_copy(msg[b], accum.at[dstr[b]], ssc[b], add=True)

    def wait_scatter(b):
        pltpu.make_async_copy(msg[b], accum.at[dstr[b]], ssc[b]).wait()

    adjust_and_gather(0, 0)

    def super_body(s_, carry):
        for b in (0, 1):
            c = 2 * s_ + b
            nb = 1 - b
            wait_gather(b)

            @pl.when(c >= 1)
            def _wait_prev():
                wait_scatter(nb)

            @pl.when(c + 1 < NCHUNK)
            def _prefetch_next():
                adjust_and_gather(c + 1, nb)

            compute(b)
            start_scatter(b)
        return carry

    lax.fori_loop(0, NCHUNK // 2, super_body, 0)
    wait_scatter(1)

    plsc.subcore_barrier()
    pltpu.sync_copy(accum.at[pl.ds(sid * ROWS_PT, ROWS_PT)],
                    out_hbm.at[cid, pl.ds(sid * ROWS_PT, ROWS_PT)])


@functools.cache
def _sc_gat_kernel():
    return pl.kernel(
        _sc_gat_edges,
        out_type=jax.ShapeDtypeStruct((H, NP, AW), jnp.float32),
        mesh=plsc.VectorSubcoreMesh(core_axis_name="c", subcore_axis_name="s",
                                    num_cores=H, num_subcores=NS),
        compiler_params=pltpu.CompilerParams(use_tc_tiling_on_sc=False,
                                             needs_layout_passes=False),
        scratch_types=[
            pltpu.VMEM((EPT,), jnp.int32),
            pltpu.VMEM((EPT,), jnp.int32),
            pltpu.VMEM((K,), jnp.int32),
            pltpu.VMEM((K,), jnp.int32),
            pltpu.VMEM((K,), jnp.int32),
            pltpu.VMEM((K,), jnp.int32),
            pltpu.VMEM((K,), jnp.int32),
            pltpu.VMEM((K,), jnp.int32),
            pltpu.VMEM((K, C), jnp.float32),
            pltpu.VMEM((K, C), jnp.float32),
            pltpu.VMEM((K, C), jnp.float32),
            pltpu.VMEM((K, C), jnp.float32),
            pltpu.VMEM((K, AW), jnp.float32),
            pltpu.VMEM((K, AW), jnp.float32),
            pltpu.VMEM((C,), jnp.float32),
            pltpu.VMEM_SHARED((NP, AW), jnp.float32),
            pltpu.SemaphoreType.DMA,
            pltpu.SemaphoreType.DMA,
            pltpu.SemaphoreType.DMA,
            pltpu.SemaphoreType.DMA,
            pltpu.SemaphoreType.DMA,
            pltpu.SemaphoreType.DMA,
        ],
    )


def _tc_in_body(x_ref, w_ref, b_ref, out_ref):
    y = jnp.dot(x_ref[...], w_ref[...].T, preferred_element_type=jnp.float32)
    y = y + b_ref[...]
    out_ref[pl.ds(0, N), :] = y[:, 0:C]
    out_ref[pl.ds(N, N), :] = y[:, C:HC]


def _tc_mid_body(acc_ref, w_ref, b_ref, out_ref):
    num0 = acc_ref[0, 0:N, 0:C]
    den0 = acc_ref[0, 0:N, C:C + 1]
    num1 = acc_ref[1, 0:N, 0:C]
    den1 = acc_ref[1, 0:N, C:C + 1]
    h0 = jnp.maximum(num0 / (den0 + 1e-16), 0.0)
    h1 = jnp.maximum(num1 / (den1 + 1e-16), 0.0)
    h = jnp.concatenate([h0, h1], axis=1)
    y = jnp.dot(h, w_ref[...].T, preferred_element_type=jnp.float32)
    y = y + b_ref[...]
    out_ref[pl.ds(0, N), :] = y[:, 0:C]
    out_ref[pl.ds(N, N), :] = y[:, C:HC]


def _tc_head_body(acc_ref, w1_ref, b1_ref, w2_ref, b2_ref, out_ref):
    num0 = acc_ref[0, 0:N, 0:C]
    den0 = acc_ref[0, 0:N, C:C + 1]
    num1 = acc_ref[1, 0:N, 0:C]
    den1 = acc_ref[1, 0:N, C:C + 1]
    h0 = jnp.maximum(num0 / (den0 + 1e-16), 0.0)
    h1 = jnp.maximum(num1 / (den1 + 1e-16), 0.0)
    h = jnp.concatenate([h0, h1], axis=1)
    y = jnp.dot(h, w1_ref[...].T, preferred_element_type=jnp.float32)
    y = y + b1_ref[...]
    y = jnp.dot(y, w2_ref[...].T, preferred_element_type=jnp.float32)
    y = y + b2_ref[...]
    m = jnp.max(y, axis=1, keepdims=True)
    z = y - m
    out_ref[...] = z - jnp.log(jnp.sum(jnp.exp(z), axis=1, keepdims=True))


_tc_in = pl.pallas_call(
    _tc_in_body,
    out_shape=jax.ShapeDtypeStruct((H * N, C), jnp.float32),
)

_tc_mid = pl.pallas_call(
    _tc_mid_body,
    out_shape=jax.ShapeDtypeStruct((H * N, C), jnp.float32),
)

_tc_head = pl.pallas_call(
    _tc_head_body,
    out_shape=jax.ShapeDtypeStruct((N, C), jnp.float32),
)


def kernel(x, edge_index, batch, W0, b0, attr0, W1, b1, attr1, Wp1, bp1, Wp2, bp2):
    src = edge_index[0]
    dst = edge_index[1]
    zeros = jnp.zeros((NP, AW), dtype=jnp.float32)
    attr0f = attr0.reshape(H, C)
    attr1f = attr1.reshape(H, C)

    sc_gat = _sc_gat_kernel()
    table0 = _tc_in(x, W0, b0.reshape(1, HC))
    acc0 = sc_gat(table0, src, dst, attr0f, zeros)
    table1 = _tc_mid(acc0, W1, b1.reshape(1, HC))
    acc1 = sc_gat(table1, src, dst, attr1f, zeros)
    return _tc_head(acc1, Wp1, bp1.reshape(1, C), Wp2, bp2.reshape(1, C))
